# baseline (speedup 1.0000x reference)
import jax
import jax.numpy as jnp
from jax.experimental import pallas as pl
from jax.experimental.pallas import tpu as pltpu

_BF = jnp.bfloat16
_F32 = jnp.float32


def _row_slab(x_ref, img, lo, th, h):
    top, bot = max(0, -lo), max(0, lo + th - h)
    core = x_ref[img, max(lo, 0):min(lo + th, h)]
    if top or bot:
        z = jnp.zeros((1,) + core.shape[1:], core.dtype)
        parts = [jnp.broadcast_to(z, (top,) + core.shape[1:])] if top else []
        parts.append(core)
        if bot:
            parts.append(jnp.broadcast_to(z, (bot,) + core.shape[1:]))
        core = jnp.concatenate(parts, axis=0)
    return core


def _conv1_kernel(x_ref, band_ref, b_ref, o_ref):
    for img in range(x_ref.shape[0]):
        slabs = []
        for kh in range(5):
            s = _row_slab(x_ref, img, kh - 2, 224, 224)
            slabs.append(jnp.pad(s, ((0, 0), (0, 32))))
        a0 = jnp.concatenate(slabs, axis=1).astype(_BF)
        acc = jnp.dot(a0, band_ref[...], preferred_element_type=_F32)
        a1 = jnp.maximum(acc + b_ref[...], 0.0)
        a1 = a1.reshape(112, 2, 3584)
        a1 = jnp.maximum(a1[:, 0], a1[:, 1])
        pooled = jnp.maximum(a1[:, :1792], a1[:, 1792:])
        o_ref[img] = pooled


def _build_band1(conv1_w):
    w1 = conv1_w.reshape(5, 5, 16)
    wi = jnp.arange(224)[:, None]
    wo = jnp.arange(224)[None, :]
    band = jnp.zeros((5, 224, 224, 16), _F32)
    for kw in range(5):
        mask = (wi == wo + kw - 2).astype(_F32)
        band = band + mask[None, :, :, None] * w1[:, kw, None, None, :]
    band = band.reshape(5, 224, 112, 2, 16).transpose(0, 1, 3, 2, 4)
    band = band.reshape(5, 224, 3584)
    band = jnp.pad(band, ((0, 0), (0, 32), (0, 0)))
    return band.reshape(1280, 3584).astype(_BF)


def _make_conv_kernel(h, w, ci, co, th, wp, out_dtype):
    def body(x_ref, w_ref, b_ref, o_ref):
        for img in range(x_ref.shape[0]):
            for rb in range(h // th):
                r0 = rb * th - 2
                pieces = []
                for kh in range(5):
                    s = _row_slab(x_ref, img, r0 + kh, th, h)
                    pieces.append(jnp.pad(s.astype(_BF),
                                          ((0, 0), (8, wp - w - 8), (0, 0))))
                a = jnp.concatenate(pieces, axis=2)
                acc = jnp.dot(a.reshape(th * wp, 5 * ci), w_ref[...],
                              preferred_element_type=_F32)
                acc3 = acc.reshape(th, wp, 640)
                conv = acc3[:, 6:6 + w, 0:co]
                for kw in range(1, 5):
                    conv = conv + acc3[:, 6 + kw:6 + kw + w,
                                       128 * kw:128 * kw + co]
                conv = jnp.maximum(conv + b_ref[...], 0.0)
                c4 = conv.reshape(th // 2, 2, w // 2, 2, co)
                rows = jnp.maximum(c4[:, 0], c4[:, 1])
                pooled = jnp.maximum(rows[:, :, 0], rows[:, :, 1])
                o_ref[img, (rb * th) // 2:(rb * th) // 2 + th // 2] = \
                    pooled.astype(o_ref.dtype)

    return body


def _conv_layer(x, w_grp, b_row, h, w, ci, co, th, wp, bn, out_dtype):
    n = x.shape[0]
    return pl.pallas_call(
        _make_conv_kernel(h, w, ci, co, th, wp, out_dtype),
        out_shape=jax.ShapeDtypeStruct((n, h // 2, w // 2, co), out_dtype),
        grid_spec=pltpu.PrefetchScalarGridSpec(
            num_scalar_prefetch=0,
            grid=(n // bn,),
            in_specs=[
                pl.BlockSpec((bn, h, w, ci), lambda b: (b, 0, 0, 0)),
                pl.BlockSpec((5 * ci, 640), lambda b: (0, 0)),
                pl.BlockSpec((1, co), lambda b: (0, 0)),
            ],
            out_specs=pl.BlockSpec((bn, h // 2, w // 2, co),
                                   lambda b: (b, 0, 0, 0)),
        ),
        compiler_params=pltpu.CompilerParams(
            dimension_semantics=("arbitrary",),
            vmem_limit_bytes=48 << 20,
        ),
    )(x, w_grp, b_row)


def _group_w(conv_w, ci, co):
    w = conv_w.reshape(5 * ci, 5, co)
    w = jnp.pad(w, ((0, 0), (0, 0), (0, 128 - co)))
    return w.reshape(5 * ci, 640).astype(_BF)


def _fc_kernel(x_ref, w_ref, b_ref, o_ref):
    o_ref[...] = (
        jnp.dot(x_ref[...], w_ref[...], preferred_element_type=_F32)
        + b_ref[...]
    )


def kernel(x, conv1_w, conv1_b, conv2_w, conv2_b, conv3_w, conv3_b,
           conv4_w, conv4_b, fc_w, fc_b):
    n = x.shape[0]
    band1 = _build_band1(conv1_w)
    b1_full = jnp.tile(conv1_b.reshape(16), 224).reshape(1, 3584)

    act1 = pl.pallas_call(
        _conv1_kernel,
        out_shape=jax.ShapeDtypeStruct((n, 112, 1792), _F32),
        grid_spec=pltpu.PrefetchScalarGridSpec(
            num_scalar_prefetch=0,
            grid=(n // 8,),
            in_specs=[
                pl.BlockSpec((8, 224, 224), lambda b: (b, 0, 0)),
                pl.BlockSpec((1280, 3584), lambda b: (0, 0)),
                pl.BlockSpec((1, 3584), lambda b: (0, 0)),
            ],
            out_specs=pl.BlockSpec((8, 112, 1792), lambda b: (b, 0, 0)),
        ),
        compiler_params=pltpu.CompilerParams(
            dimension_semantics=("arbitrary",),
            vmem_limit_bytes=48 << 20,
        ),
    )(x[:, 0], band1, b1_full)

    act2 = _conv_layer(act1.reshape(n, 112, 112, 16),
                       _group_w(conv2_w, 16, 32), conv2_b,
                       112, 112, 16, 32, th=16, wp=128, bn=2, out_dtype=_F32)
    act3 = _conv_layer(act2, _group_w(conv3_w, 32, 64), conv3_b,
                       56, 56, 32, 64, th=28, wp=72, bn=8, out_dtype=_F32)
    act4 = _conv_layer(act3, _group_w(conv4_w, 64, 128), conv4_b,
                       28, 28, 64, 128, th=28, wp=48, bn=8, out_dtype=_BF)

    out = pl.pallas_call(
        _fc_kernel,
        out_shape=jax.ShapeDtypeStruct((n, 15), _F32),
        grid_spec=pltpu.PrefetchScalarGridSpec(
            num_scalar_prefetch=0,
            grid=(2,),
            in_specs=[
                pl.BlockSpec((n // 2, 25088), lambda i: (i, 0)),
                pl.BlockSpec((25088, 15), lambda i: (0, 0)),
                pl.BlockSpec((1, 15), lambda i: (0, 0)),
            ],
            out_specs=pl.BlockSpec((n // 2, 15), lambda i: (i, 0)),
        ),
        compiler_params=pltpu.CompilerParams(
            dimension_semantics=("arbitrary",),
            vmem_limit_bytes=48 << 20,
        ),
    )(act4.reshape(n, 25088), fc_w.astype(_BF), fc_b)
    return out


# speedup vs baseline: 1.0152x; 1.0152x over previous
import jax
import jax.numpy as jnp
from jax.experimental import pallas as pl
from jax.experimental.pallas import tpu as pltpu

_BF = jnp.bfloat16
_F32 = jnp.float32


def _row_slab(x_ref, img, lo, th, h):
    top, bot = max(0, -lo), max(0, lo + th - h)
    core = x_ref[img, max(lo, 0):min(lo + th, h)]
    if top or bot:
        z = jnp.zeros((1,) + core.shape[1:], core.dtype)
        parts = [jnp.broadcast_to(z, (top,) + core.shape[1:])] if top else []
        parts.append(core)
        if bot:
            parts.append(jnp.broadcast_to(z, (bot,) + core.shape[1:]))
        core = jnp.concatenate(parts, axis=0)
    return core


def _conv1_kernel(x_ref, band_ref, b_ref, o_ref):
    for img in range(x_ref.shape[0]):
        slabs = []
        for kh in range(5):
            s = _row_slab(x_ref, img, kh - 2, 224, 224)
            slabs.append(jnp.pad(s, ((0, 0), (0, 32))))
        a0 = jnp.concatenate(slabs, axis=1).astype(_BF)
        acc = jnp.dot(a0, band_ref[...], preferred_element_type=_F32)
        a1 = jnp.maximum(acc + b_ref[...], 0.0)
        a1 = a1.reshape(112, 2, 3584)
        a1 = jnp.maximum(a1[:, 0], a1[:, 1])
        pooled = jnp.maximum(a1[:, :1792], a1[:, 1792:])
        o_ref[img] = pooled


def _build_band1(conv1_w):
    w1 = conv1_w.reshape(5, 5, 16)
    wi = jnp.arange(224)[:, None]
    wo = jnp.arange(224)[None, :]
    band = jnp.zeros((5, 224, 224, 16), _F32)
    for kw in range(5):
        mask = (wi == wo + kw - 2).astype(_F32)
        band = band + mask[None, :, :, None] * w1[:, kw, None, None, :]
    band = band.reshape(5, 224, 112, 2, 16).transpose(0, 1, 3, 2, 4)
    band = band.reshape(5, 224, 3584)
    band = jnp.pad(band, ((0, 0), (0, 32), (0, 0)))
    return band.reshape(1280, 3584).astype(_BF)


def _make_conv_kernel(h, w, ci, co, th, wp, out_dtype):
    def body(x_ref, w_ref, b_ref, o_ref):
        for img in range(x_ref.shape[0]):
            for rb in range(h // th):
                r0 = rb * th - 2
                pieces = []
                for kh in range(5):
                    s = _row_slab(x_ref, img, r0 + kh, th, h)
                    pieces.append(jnp.pad(s.astype(_BF),
                                          ((0, 0), (8, wp - w - 8), (0, 0))))
                a = jnp.concatenate(pieces, axis=2)
                acc = jnp.dot(a.reshape(th * wp, 5 * ci), w_ref[...],
                              preferred_element_type=_F32)
                acc3 = acc.reshape(th, wp, 640)
                conv = acc3[:, 6:6 + w, 0:co]
                for kw in range(1, 5):
                    conv = conv + acc3[:, 6 + kw:6 + kw + w,
                                       128 * kw:128 * kw + co]
                conv = jnp.maximum(conv + b_ref[...], 0.0)
                c4 = conv.reshape(th // 2, 2, w // 2, 2, co)
                rows = jnp.maximum(c4[:, 0], c4[:, 1])
                pooled = jnp.maximum(rows[:, :, 0], rows[:, :, 1])
                o_ref[img, (rb * th) // 2:(rb * th) // 2 + th // 2] = \
                    pooled.astype(o_ref.dtype)

    return body


def _conv_layer(x, w_grp, b_row, h, w, ci, co, th, wp, bn, out_dtype):
    n = x.shape[0]
    return pl.pallas_call(
        _make_conv_kernel(h, w, ci, co, th, wp, out_dtype),
        out_shape=jax.ShapeDtypeStruct((n, h // 2, w // 2, co), out_dtype),
        grid_spec=pltpu.PrefetchScalarGridSpec(
            num_scalar_prefetch=0,
            grid=(n // bn,),
            in_specs=[
                pl.BlockSpec((bn, h, w, ci), lambda b: (b, 0, 0, 0)),
                pl.BlockSpec((5 * ci, 640), lambda b: (0, 0)),
                pl.BlockSpec((1, co), lambda b: (0, 0)),
            ],
            out_specs=pl.BlockSpec((bn, h // 2, w // 2, co),
                                   lambda b: (b, 0, 0, 0)),
        ),
        compiler_params=pltpu.CompilerParams(
            dimension_semantics=("arbitrary",),
            vmem_limit_bytes=48 << 20,
        ),
    )(x, w_grp, b_row)


def _group_w(conv_w, ci, co):
    w = conv_w.reshape(5 * ci, 5, co)
    w = jnp.pad(w, ((0, 0), (0, 0), (0, 128 - co)))
    return w.reshape(5 * ci, 640).astype(_BF)


def _fc_kernel(x_ref, w_ref, b_ref, o_ref):
    o_ref[...] = (
        jnp.dot(x_ref[...], w_ref[...], preferred_element_type=_F32)
        + b_ref[...]
    )


def kernel(x, conv1_w, conv1_b, conv2_w, conv2_b, conv3_w, conv3_b,
           conv4_w, conv4_b, fc_w, fc_b):
    n = x.shape[0]
    band1 = _build_band1(conv1_w)
    b1_full = jnp.tile(conv1_b.reshape(16), 224).reshape(1, 3584)

    act1 = pl.pallas_call(
        _conv1_kernel,
        out_shape=jax.ShapeDtypeStruct((n, 112, 1792), _F32),
        grid_spec=pltpu.PrefetchScalarGridSpec(
            num_scalar_prefetch=0,
            grid=(n // 8,),
            in_specs=[
                pl.BlockSpec((8, 224, 224), lambda b: (b, 0, 0)),
                pl.BlockSpec((1280, 3584), lambda b: (0, 0)),
                pl.BlockSpec((1, 3584), lambda b: (0, 0)),
            ],
            out_specs=pl.BlockSpec((8, 112, 1792), lambda b: (b, 0, 0)),
        ),
        compiler_params=pltpu.CompilerParams(
            dimension_semantics=("arbitrary",),
            vmem_limit_bytes=48 << 20,
        ),
    )(x[:, 0], band1, b1_full)

    act2 = _conv_layer(act1.reshape(n, 112, 112, 16),
                       _group_w(conv2_w, 16, 32), conv2_b,
                       112, 112, 16, 32, th=28, wp=128, bn=2, out_dtype=_F32)
    act3 = _conv_layer(act2, _group_w(conv3_w, 32, 64), conv3_b,
                       56, 56, 32, 64, th=28, wp=72, bn=8, out_dtype=_F32)
    act4 = _conv_layer(act3, _group_w(conv4_w, 64, 128), conv4_b,
                       28, 28, 64, 128, th=28, wp=48, bn=8, out_dtype=_BF)

    out = pl.pallas_call(
        _fc_kernel,
        out_shape=jax.ShapeDtypeStruct((n, 15), _F32),
        grid_spec=pltpu.PrefetchScalarGridSpec(
            num_scalar_prefetch=0,
            grid=(2,),
            in_specs=[
                pl.BlockSpec((n // 2, 25088), lambda i: (i, 0)),
                pl.BlockSpec((25088, 15), lambda i: (0, 0)),
                pl.BlockSpec((1, 15), lambda i: (0, 0)),
            ],
            out_specs=pl.BlockSpec((n // 2, 15), lambda i: (i, 0)),
        ),
        compiler_params=pltpu.CompilerParams(
            dimension_semantics=("arbitrary",),
            vmem_limit_bytes=48 << 20,
        ),
    )(act4.reshape(n, 25088), fc_w.astype(_BF), fc_b)
    return out


# speedup vs baseline: 1.0213x; 1.0061x over previous
import jax
import jax.numpy as jnp
from jax.experimental import pallas as pl
from jax.experimental.pallas import tpu as pltpu

_BF = jnp.bfloat16
_F32 = jnp.float32


def _row_slab(x_ref, img, lo, th, h):
    top, bot = max(0, -lo), max(0, lo + th - h)
    core = x_ref[img, max(lo, 0):min(lo + th, h)]
    if top or bot:
        z = jnp.zeros((1,) + core.shape[1:], core.dtype)
        parts = [jnp.broadcast_to(z, (top,) + core.shape[1:])] if top else []
        parts.append(core)
        if bot:
            parts.append(jnp.broadcast_to(z, (bot,) + core.shape[1:]))
        core = jnp.concatenate(parts, axis=0)
    return core


def _conv1_kernel(x_ref, band_ref, b_ref, o_ref):
    for img in range(x_ref.shape[0]):
        slabs = []
        for kh in range(5):
            s = _row_slab(x_ref, img, kh - 2, 224, 224)
            slabs.append(jnp.pad(s, ((0, 0), (0, 32))))
        a0 = jnp.concatenate(slabs, axis=1).astype(_BF)
        acc = jnp.dot(a0, band_ref[...], preferred_element_type=_F32)
        a1 = jnp.maximum(acc + b_ref[...], 0.0)
        a1 = a1.reshape(112, 2, 3584)
        a1 = jnp.maximum(a1[:, 0], a1[:, 1])
        pooled = jnp.maximum(a1[:, :1792], a1[:, 1792:])
        o_ref[img] = pooled


def _build_band1(conv1_w):
    w1 = conv1_w.reshape(5, 5, 16)
    wi = jnp.arange(224)[:, None]
    wo = jnp.arange(224)[None, :]
    band = jnp.zeros((5, 224, 224, 16), _F32)
    for kw in range(5):
        mask = (wi == wo + kw - 2).astype(_F32)
        band = band + mask[None, :, :, None] * w1[:, kw, None, None, :]
    band = band.reshape(5, 224, 112, 2, 16).transpose(0, 1, 3, 2, 4)
    band = band.reshape(5, 224, 3584)
    band = jnp.pad(band, ((0, 0), (0, 32), (0, 0)))
    return band.reshape(1280, 3584).astype(_BF)


def _make_conv_kernel(h, w, ci, co, th, wp, out_dtype):
    def body(x_ref, w_ref, b_ref, o_ref):
        for img in range(x_ref.shape[0]):
            for rb in range(h // th):
                r0 = rb * th - 2
                pieces = []
                for kh in range(5):
                    s = _row_slab(x_ref, img, r0 + kh, th, h)
                    pieces.append(jnp.pad(s.astype(_BF),
                                          ((0, 0), (8, wp - w - 8), (0, 0))))
                a = jnp.concatenate(pieces, axis=2)
                acc = jnp.dot(a.reshape(th * wp, 5 * ci), w_ref[...],
                              preferred_element_type=_F32)
                acc3 = acc.reshape(th, wp, 640)
                conv = acc3[:, 6:6 + w, 0:co]
                for kw in range(1, 5):
                    conv = conv + acc3[:, 6 + kw:6 + kw + w,
                                       128 * kw:128 * kw + co]
                conv = jnp.maximum(conv + b_ref[...], 0.0)
                c4 = conv.reshape(th // 2, 2, w // 2, 2, co)
                rows = jnp.maximum(c4[:, 0], c4[:, 1])
                pooled = jnp.maximum(rows[:, :, 0], rows[:, :, 1])
                o_ref[img, (rb * th) // 2:(rb * th) // 2 + th // 2] = \
                    pooled.astype(o_ref.dtype)

    return body


def _conv_layer(x, w_grp, b_row, h, w, ci, co, th, wp, bn, out_dtype):
    n = x.shape[0]
    return pl.pallas_call(
        _make_conv_kernel(h, w, ci, co, th, wp, out_dtype),
        out_shape=jax.ShapeDtypeStruct((n, h // 2, w // 2, co), out_dtype),
        grid_spec=pltpu.PrefetchScalarGridSpec(
            num_scalar_prefetch=0,
            grid=(n // bn,),
            in_specs=[
                pl.BlockSpec((bn, h, w, ci), lambda b: (b, 0, 0, 0)),
                pl.BlockSpec((5 * ci, 640), lambda b: (0, 0)),
                pl.BlockSpec((1, co), lambda b: (0, 0)),
            ],
            out_specs=pl.BlockSpec((bn, h // 2, w // 2, co),
                                   lambda b: (b, 0, 0, 0)),
        ),
        compiler_params=pltpu.CompilerParams(
            dimension_semantics=("arbitrary",),
            vmem_limit_bytes=56 << 20,
        ),
    )(x, w_grp, b_row)


def _group_w(conv_w, ci, co):
    w = conv_w.reshape(5 * ci, 5, co)
    w = jnp.pad(w, ((0, 0), (0, 0), (0, 128 - co)))
    return w.reshape(5 * ci, 640).astype(_BF)


def _fc_kernel(x_ref, w_ref, b_ref, o_ref):
    o_ref[...] = (
        jnp.dot(x_ref[...], w_ref[...], preferred_element_type=_F32)
        + b_ref[...]
    )


def kernel(x, conv1_w, conv1_b, conv2_w, conv2_b, conv3_w, conv3_b,
           conv4_w, conv4_b, fc_w, fc_b):
    n = x.shape[0]
    band1 = _build_band1(conv1_w)
    b1_full = jnp.tile(conv1_b.reshape(16), 224).reshape(1, 3584)

    act1 = pl.pallas_call(
        _conv1_kernel,
        out_shape=jax.ShapeDtypeStruct((n, 112, 1792), _F32),
        grid_spec=pltpu.PrefetchScalarGridSpec(
            num_scalar_prefetch=0,
            grid=(n // 8,),
            in_specs=[
                pl.BlockSpec((8, 224, 224), lambda b: (b, 0, 0)),
                pl.BlockSpec((1280, 3584), lambda b: (0, 0)),
                pl.BlockSpec((1, 3584), lambda b: (0, 0)),
            ],
            out_specs=pl.BlockSpec((8, 112, 1792), lambda b: (b, 0, 0)),
        ),
        compiler_params=pltpu.CompilerParams(
            dimension_semantics=("arbitrary",),
            vmem_limit_bytes=56 << 20,
        ),
    )(x[:, 0], band1, b1_full)

    act2 = _conv_layer(act1.reshape(n, 112, 112, 16),
                       _group_w(conv2_w, 16, 32), conv2_b,
                       112, 112, 16, 32, th=56, wp=128, bn=1, out_dtype=_F32)
    act3 = _conv_layer(act2, _group_w(conv3_w, 32, 64), conv3_b,
                       56, 56, 32, 64, th=28, wp=72, bn=8, out_dtype=_F32)
    act4 = _conv_layer(act3, _group_w(conv4_w, 64, 128), conv4_b,
                       28, 28, 64, 128, th=28, wp=48, bn=8, out_dtype=_BF)

    out = pl.pallas_call(
        _fc_kernel,
        out_shape=jax.ShapeDtypeStruct((n, 15), _F32),
        grid_spec=pltpu.PrefetchScalarGridSpec(
            num_scalar_prefetch=0,
            grid=(2,),
            in_specs=[
                pl.BlockSpec((n // 2, 25088), lambda i: (i, 0)),
                pl.BlockSpec((25088, 15), lambda i: (0, 0)),
                pl.BlockSpec((1, 15), lambda i: (0, 0)),
            ],
            out_specs=pl.BlockSpec((n // 2, 15), lambda i: (i, 0)),
        ),
        compiler_params=pltpu.CompilerParams(
            dimension_semantics=("arbitrary",),
            vmem_limit_bytes=56 << 20,
        ),
    )(act4.reshape(n, 25088), fc_w.astype(_BF), fc_b)
    return out


# speedup vs baseline: 1.0234x; 1.0020x over previous
import jax
import jax.numpy as jnp
from jax.experimental import pallas as pl
from jax.experimental.pallas import tpu as pltpu

_BF = jnp.bfloat16
_F32 = jnp.float32


def _row_slab(x_ref, img, lo, th, h):
    top, bot = max(0, -lo), max(0, lo + th - h)
    core = x_ref[img, max(lo, 0):min(lo + th, h)]
    if top or bot:
        z = jnp.zeros((1,) + core.shape[1:], core.dtype)
        parts = [jnp.broadcast_to(z, (top,) + core.shape[1:])] if top else []
        parts.append(core)
        if bot:
            parts.append(jnp.broadcast_to(z, (bot,) + core.shape[1:]))
        core = jnp.concatenate(parts, axis=0)
    return core


def _conv1_kernel(x_ref, band_ref, b_ref, o_ref):
    for img in range(x_ref.shape[0]):
        slabs = []
        for kh in range(5):
            s = _row_slab(x_ref, img, kh - 2, 224, 224)
            slabs.append(jnp.pad(s, ((0, 0), (0, 32))))
        a0 = jnp.concatenate(slabs, axis=1).astype(_BF)
        acc = jnp.dot(a0, band_ref[...], preferred_element_type=_F32)
        a1 = jnp.maximum(acc + b_ref[...], 0.0)
        a1 = a1.reshape(112, 2, 3584)
        a1 = jnp.maximum(a1[:, 0], a1[:, 1])
        pooled = jnp.maximum(a1[:, :1792], a1[:, 1792:])
        o_ref[img] = pooled


def _build_band1(conv1_w):
    w1 = conv1_w.reshape(5, 5, 16)
    wi = jnp.arange(224)[:, None]
    wo = jnp.arange(224)[None, :]
    band = jnp.zeros((5, 224, 224, 16), _F32)
    for kw in range(5):
        mask = (wi == wo + kw - 2).astype(_F32)
        band = band + mask[None, :, :, None] * w1[:, kw, None, None, :]
    band = band.reshape(5, 224, 112, 2, 16).transpose(0, 1, 3, 2, 4)
    band = band.reshape(5, 224, 3584)
    band = jnp.pad(band, ((0, 0), (0, 32), (0, 0)))
    return band.reshape(1280, 3584).astype(_BF)


def _make_conv_kernel(h, w, ci, co, th, wp, out_dtype):
    def body(x_ref, w_ref, b_ref, o_ref):
        for img in range(x_ref.shape[0]):
            for rb in range(h // th):
                r0 = rb * th - 2
                pieces = []
                for kh in range(5):
                    s = _row_slab(x_ref, img, r0 + kh, th, h)
                    pieces.append(jnp.pad(s.astype(_BF),
                                          ((0, 0), (8, wp - w - 8), (0, 0))))
                a = jnp.concatenate(pieces, axis=2)
                acc = jnp.dot(a.reshape(th * wp, 5 * ci), w_ref[...],
                              preferred_element_type=_F32)
                acc3 = acc.reshape(th, wp, 640)
                conv = acc3[:, 6:6 + w, 0:co]
                for kw in range(1, 5):
                    conv = conv + acc3[:, 6 + kw:6 + kw + w,
                                       128 * kw:128 * kw + co]
                conv = jnp.maximum(conv + b_ref[...], 0.0)
                c4 = conv.reshape(th // 2, 2, w // 2, 2, co)
                rows = jnp.maximum(c4[:, 0], c4[:, 1])
                pooled = jnp.maximum(rows[:, :, 0], rows[:, :, 1])
                o_ref[img, (rb * th) // 2:(rb * th) // 2 + th // 2] = \
                    pooled.astype(o_ref.dtype)

    return body


def _conv_layer(x, w_grp, b_row, h, w, ci, co, th, wp, bn, out_dtype):
    n = x.shape[0]
    return pl.pallas_call(
        _make_conv_kernel(h, w, ci, co, th, wp, out_dtype),
        out_shape=jax.ShapeDtypeStruct((n, h // 2, w // 2, co), out_dtype),
        grid_spec=pltpu.PrefetchScalarGridSpec(
            num_scalar_prefetch=0,
            grid=(n // bn,),
            in_specs=[
                pl.BlockSpec((bn, h, w, ci), lambda b: (b, 0, 0, 0)),
                pl.BlockSpec((5 * ci, 640), lambda b: (0, 0)),
                pl.BlockSpec((1, co), lambda b: (0, 0)),
            ],
            out_specs=pl.BlockSpec((bn, h // 2, w // 2, co),
                                   lambda b: (b, 0, 0, 0)),
        ),
        compiler_params=pltpu.CompilerParams(
            dimension_semantics=("arbitrary",),
            vmem_limit_bytes=56 << 20,
        ),
    )(x, w_grp, b_row)


def _group_w(conv_w, ci, co):
    w = conv_w.reshape(5 * ci, 5, co)
    w = jnp.pad(w, ((0, 0), (0, 0), (0, 128 - co)))
    return w.reshape(5 * ci, 640).astype(_BF)


def _fc_kernel(x_ref, w_ref, b_ref, o_ref):
    o_ref[...] = (
        jnp.dot(x_ref[...], w_ref[...], preferred_element_type=_F32)
        + b_ref[...]
    )


def kernel(x, conv1_w, conv1_b, conv2_w, conv2_b, conv3_w, conv3_b,
           conv4_w, conv4_b, fc_w, fc_b):
    n = x.shape[0]
    band1 = _build_band1(conv1_w)
    b1_full = jnp.tile(conv1_b.reshape(16), 224).reshape(1, 3584)

    act1 = pl.pallas_call(
        _conv1_kernel,
        out_shape=jax.ShapeDtypeStruct((n, 112, 1792), _F32),
        grid_spec=pltpu.PrefetchScalarGridSpec(
            num_scalar_prefetch=0,
            grid=(n // 8,),
            in_specs=[
                pl.BlockSpec((8, 224, 224), lambda b: (b, 0, 0)),
                pl.BlockSpec((1280, 3584), lambda b: (0, 0)),
                pl.BlockSpec((1, 3584), lambda b: (0, 0)),
            ],
            out_specs=pl.BlockSpec((8, 112, 1792), lambda b: (b, 0, 0)),
        ),
        compiler_params=pltpu.CompilerParams(
            dimension_semantics=("arbitrary",),
            vmem_limit_bytes=56 << 20,
        ),
    )(x[:, 0], band1, b1_full)

    act2 = _conv_layer(act1.reshape(n, 112, 112, 16),
                       _group_w(conv2_w, 16, 32), conv2_b,
                       112, 112, 16, 32, th=56, wp=128, bn=1, out_dtype=_F32)
    act3 = _conv_layer(act2, _group_w(conv3_w, 32, 64), conv3_b,
                       56, 56, 32, 64, th=56, wp=72, bn=4, out_dtype=_F32)
    act4 = _conv_layer(act3, _group_w(conv4_w, 64, 128), conv4_b,
                       28, 28, 64, 128, th=28, wp=48, bn=8, out_dtype=_BF)

    out = pl.pallas_call(
        _fc_kernel,
        out_shape=jax.ShapeDtypeStruct((n, 15), _F32),
        grid_spec=pltpu.PrefetchScalarGridSpec(
            num_scalar_prefetch=0,
            grid=(2,),
            in_specs=[
                pl.BlockSpec((n // 2, 25088), lambda i: (i, 0)),
                pl.BlockSpec((25088, 15), lambda i: (0, 0)),
                pl.BlockSpec((1, 15), lambda i: (0, 0)),
            ],
            out_specs=pl.BlockSpec((n // 2, 15), lambda i: (i, 0)),
        ),
        compiler_params=pltpu.CompilerParams(
            dimension_semantics=("arbitrary",),
            vmem_limit_bytes=56 << 20,
        ),
    )(act4.reshape(n, 25088), fc_w.astype(_BF), fc_b)
    return out
